# baseline (speedup 1.0000x reference)
import functools

import jax
import jax.numpy as jnp
from jax.experimental import pallas as pl
from jax.experimental.pallas import tpu as pltpu

N = 10000
E = 320000
H = 128
S = 16
SH = 128

EBLK = 2560
NBLK = 2000


def _prep_body(h_ref, w1a_ref, w1b_ref, wv_ref, a_ref, b_ref, hv_ref):
    h = h_ref[...]
    a_ref[...] = jnp.dot(h, w1a_ref[...], preferred_element_type=jnp.float32)
    b_ref[...] = jnp.dot(h, w1b_ref[...], preferred_element_type=jnp.float32)
    hv_ref[...] = jnp.dot(h, wv_ref[...], preferred_element_type=jnp.float32)


def _prep(h, w1a, w1b, wv):
    grid = N // NBLK
    return pl.pallas_call(
        _prep_body,
        grid=(grid,),
        in_specs=[
            pl.BlockSpec((NBLK, H), lambda i: (i, 0)),
            pl.BlockSpec((H, SH), lambda i: (0, 0)),
            pl.BlockSpec((H, SH), lambda i: (0, 0)),
            pl.BlockSpec((H, H), lambda i: (0, 0)),
        ],
        out_specs=[
            pl.BlockSpec((NBLK, SH), lambda i: (i, 0)),
            pl.BlockSpec((NBLK, SH), lambda i: (i, 0)),
            pl.BlockSpec((NBLK, H), lambda i: (i, 0)),
        ],
        out_shape=[
            jax.ShapeDtypeStruct((N, SH), jnp.float32),
            jax.ShapeDtypeStruct((N, SH), jnp.float32),
            jax.ShapeDtypeStruct((N, H), jnp.float32),
        ],
    )(h, w1a, w1b, wv)


def _score_body(g_ref, rp_ref, w1r_ref, w2_ref, b2_ref, w3_ref, b3_ref, s_ref):
    g = g_ref[...]
    rp = rp_ref[...]
    x1 = g + jnp.dot(rp, w1r_ref[...], preferred_element_type=jnp.float32)
    x1 = x1 * jax.nn.sigmoid(x1)
    x2 = jnp.dot(x1, w2_ref[...], preferred_element_type=jnp.float32) + b2_ref[...]
    x2 = x2 * jax.nn.sigmoid(x2)
    s = jnp.sum(x2 * w3_ref[...], axis=1, keepdims=True) + b3_ref[0, 0]
    s_ref[...] = s


def _scores(g, rp_ext, w1r_ext, w2, b2, w3, b3):
    grid = E // EBLK
    return pl.pallas_call(
        _score_body,
        grid=(grid,),
        in_specs=[
            pl.BlockSpec((EBLK, SH), lambda i: (i, 0)),
            pl.BlockSpec((EBLK, 32), lambda i: (i, 0)),
            pl.BlockSpec((32, SH), lambda i: (0, 0)),
            pl.BlockSpec((SH, SH), lambda i: (0, 0)),
            pl.BlockSpec((1, SH), lambda i: (0, 0)),
            pl.BlockSpec((1, SH), lambda i: (0, 0)),
            pl.BlockSpec((1, 1), lambda i: (0, 0)),
        ],
        out_specs=pl.BlockSpec((EBLK, 1), lambda i: (i, 0)),
        out_shape=jax.ShapeDtypeStruct((E, 1), jnp.float32),
    )(g, rp_ext, w1r_ext, w2, b2, w3, b3)


def kernel(h, edge_index, rel_pos, distance, W1, b1, W2, b2, W3, b3, Wv):
    dst = edge_index[0]
    src = edge_index[1]

    a, b, hv = _prep(h, W1[:H], W1[H:2 * H], Wv)

    ones = jnp.ones((E, 1), jnp.float32)
    zeros = jnp.zeros((E, 32 - S - 2), jnp.float32)
    rp_ext = jnp.concatenate([rel_pos, distance, ones, zeros], axis=1)
    w1r_ext = jnp.concatenate(
        [W1[2 * H:], b1[None, :], jnp.zeros((32 - S - 2, SH), jnp.float32)], axis=0)

    g = a[dst] + b[src]
    scores = _scores(g, rp_ext, w1r_ext, W2, b2[None, :], W3.T, b3[None, :])[:, 0]

    max_per_dst = jax.ops.segment_max(scores, dst, num_segments=N)
    shifted = scores - max_per_dst[dst]
    exp_scores = jnp.exp(shifted)
    den = jax.ops.segment_sum(exp_scores, dst, num_segments=N)
    alpha = exp_scores / (den[dst] + 1e-12)

    msg = hv[src] * alpha[:, None]
    out = jnp.zeros_like(h).at[dst].add(msg)
    return h + out


# speedup vs baseline: 1.1801x; 1.1801x over previous
import functools

import jax
import jax.numpy as jnp
from jax import lax
from jax.experimental import pallas as pl
from jax.experimental.pallas import tpu as pltpu
from jax.experimental.pallas import tpu_sc as plsc

N = 10000
E = 320000
H = 128
S = 16
SH = 128

EBLK = 2560
NBLK = 2000

NW = 32
EPW = E // NW
GK = 80
GCH = EPW // GK


def _sc_mesh():
    return plsc.VectorSubcoreMesh(core_axis_name="c", subcore_axis_name="s")


def _gather_body(a_hbm, b_hbm, dst_hbm, src_hbm, ga_hbm, gb_hbm,
                 idx_v, rows_v, sem):
    wid = lax.axis_index("s") * 2 + lax.axis_index("c")

    def chunk(c, _):
        base = wid * EPW + c * GK
        pltpu.sync_copy(dst_hbm.at[pl.ds(base, GK)], idx_v)
        pltpu.async_copy(a_hbm.at[idx_v], rows_v, sem).wait()
        pltpu.sync_copy(rows_v, ga_hbm.at[pl.ds(base, GK)])
        pltpu.sync_copy(src_hbm.at[pl.ds(base, GK)], idx_v)
        pltpu.async_copy(b_hbm.at[idx_v], rows_v, sem).wait()
        pltpu.sync_copy(rows_v, gb_hbm.at[pl.ds(base, GK)])
        return ()

    lax.fori_loop(0, GCH, chunk, (), unroll=False)


def _sc_gather(a, b, dst, src):
    k = pl.kernel(
        _gather_body,
        mesh=_sc_mesh(),
        out_type=[
            jax.ShapeDtypeStruct((E, SH), jnp.float32),
            jax.ShapeDtypeStruct((E, SH), jnp.float32),
        ],
        scratch_types=[
            pltpu.VMEM((GK,), jnp.int32),
            pltpu.VMEM((GK, SH), jnp.float32),
            pltpu.SemaphoreType.DMA,
        ],
    )
    return k(a, b, dst, src)


def _prep_body(h_ref, w1a_ref, w1b_ref, wv_ref, a_ref, b_ref, hv_ref):
    h = h_ref[...]
    a_ref[...] = jnp.dot(h, w1a_ref[...], preferred_element_type=jnp.float32)
    b_ref[...] = jnp.dot(h, w1b_ref[...], preferred_element_type=jnp.float32)
    hv_ref[...] = jnp.dot(h, wv_ref[...], preferred_element_type=jnp.float32)


def _prep(h, w1a, w1b, wv):
    grid = N // NBLK
    return pl.pallas_call(
        _prep_body,
        grid=(grid,),
        in_specs=[
            pl.BlockSpec((NBLK, H), lambda i: (i, 0)),
            pl.BlockSpec((H, SH), lambda i: (0, 0)),
            pl.BlockSpec((H, SH), lambda i: (0, 0)),
            pl.BlockSpec((H, H), lambda i: (0, 0)),
        ],
        out_specs=[
            pl.BlockSpec((NBLK, SH), lambda i: (i, 0)),
            pl.BlockSpec((NBLK, SH), lambda i: (i, 0)),
            pl.BlockSpec((NBLK, H), lambda i: (i, 0)),
        ],
        out_shape=[
            jax.ShapeDtypeStruct((N, SH), jnp.float32),
            jax.ShapeDtypeStruct((N, SH), jnp.float32),
            jax.ShapeDtypeStruct((N, H), jnp.float32),
        ],
    )(h, w1a, w1b, wv)


def _score_body(ga_ref, gb_ref, rp_ref, w1r_ref, w2_ref, b2_ref, w3_ref, b3_ref, s_ref):
    g = ga_ref[...] + gb_ref[...]
    rp = rp_ref[...]
    x1 = g + jnp.dot(rp, w1r_ref[...], preferred_element_type=jnp.float32)
    x1 = x1 * jax.nn.sigmoid(x1)
    x2 = jnp.dot(x1, w2_ref[...], preferred_element_type=jnp.float32) + b2_ref[...]
    x2 = x2 * jax.nn.sigmoid(x2)
    s = jnp.sum(x2 * w3_ref[...], axis=1, keepdims=True) + b3_ref[0, 0]
    s_ref[...] = s


def _scores(ga, gb, rp_ext, w1r_ext, w2, b2, w3, b3):
    grid = E // EBLK
    return pl.pallas_call(
        _score_body,
        grid=(grid,),
        in_specs=[
            pl.BlockSpec((EBLK, SH), lambda i: (i, 0)),
            pl.BlockSpec((EBLK, SH), lambda i: (i, 0)),
            pl.BlockSpec((EBLK, 32), lambda i: (i, 0)),
            pl.BlockSpec((32, SH), lambda i: (0, 0)),
            pl.BlockSpec((SH, SH), lambda i: (0, 0)),
            pl.BlockSpec((1, SH), lambda i: (0, 0)),
            pl.BlockSpec((1, SH), lambda i: (0, 0)),
            pl.BlockSpec((1, 1), lambda i: (0, 0)),
        ],
        out_specs=pl.BlockSpec((EBLK, 1), lambda i: (i, 0)),
        out_shape=jax.ShapeDtypeStruct((E, 1), jnp.float32),
    )(ga, gb, rp_ext, w1r_ext, w2, b2, w3, b3)


def kernel(h, edge_index, rel_pos, distance, W1, b1, W2, b2, W3, b3, Wv):
    dst = edge_index[0]
    src = edge_index[1]

    a, b, hv = _prep(h, W1[:H], W1[H:2 * H], Wv)

    ones = jnp.ones((E, 1), jnp.float32)
    zeros = jnp.zeros((E, 32 - S - 2), jnp.float32)
    rp_ext = jnp.concatenate([rel_pos, distance, ones, zeros], axis=1)
    w1r_ext = jnp.concatenate(
        [W1[2 * H:], b1[None, :], jnp.zeros((32 - S - 2, SH), jnp.float32)], axis=0)

    ga, gb = _sc_gather(a, b, dst, src)
    scores = _scores(ga, gb, rp_ext, w1r_ext, W2, b2[None, :], W3.T, b3[None, :])[:, 0]

    max_per_dst = jax.ops.segment_max(scores, dst, num_segments=N)
    shifted = scores - max_per_dst[dst]
    exp_scores = jnp.exp(shifted)
    den = jax.ops.segment_sum(exp_scores, dst, num_segments=N)
    alpha = exp_scores / (den[dst] + 1e-12)

    msg = hv[src] * alpha[:, None]
    out = jnp.zeros_like(h).at[dst].add(msg)
    return h + out
